# Initial kernel scaffold; baseline (speedup 1.0000x reference)
import functools

import jax
import jax.numpy as jnp
from jax import lax
from jax.experimental import pallas as pl
from jax.experimental.pallas import tpu as pltpu
from jax.experimental.pallas import tpu_sc as plsc

N = 10000
E = 160000
DIN = 256
H = 8
DH = 32
HD = H * DH

BN = 400
NCHUNK = E // 128
NC = 2
NS = 16
ROWS_PER_TILE = N // NS


def _project_body(x_ref, w_ref, sd_ref, xlp_ref, scores_ref, gmax_ref):
    i = pl.program_id(0)
    y = lax.dot_general(x_ref[...], w_ref[...],
                        dimension_numbers=(((1,), (1,)), ((), ())),
                        preferred_element_type=jnp.float32)
    scores = jnp.dot(y, sd_ref[...], preferred_element_type=jnp.float32)
    xlp_ref[...] = jnp.stack([y[:, :128], y[:, 128:]], axis=0)
    scores_ref[...] = scores
    m = jnp.max(scores[:, :H], axis=0)
    g16 = jnp.concatenate([m, m])[None, :]

    @pl.when(i == 0)
    def _():
        gmax_ref[...] = g16

    @pl.when(i != 0)
    def _():
        gmax_ref[...] = jnp.maximum(gmax_ref[...], g16)


def _project(x, W, SD):
    return pl.pallas_call(
        _project_body,
        grid=(N // BN,),
        in_specs=[
            pl.BlockSpec((BN, DIN), lambda i: (i, 0)),
            pl.BlockSpec((HD, DIN), lambda i: (0, 0)),
            pl.BlockSpec((DIN, 16), lambda i: (0, 0)),
        ],
        out_specs=[
            pl.BlockSpec((2, BN, 128), lambda i: (0, i, 0)),
            pl.BlockSpec((BN, 16), lambda i: (i, 0)),
            pl.BlockSpec((1, 16), lambda i: (0, 0)),
        ],
        out_shape=[
            jax.ShapeDtypeStruct((2, N, 128), jnp.float32),
            jax.ShapeDtypeStruct((N, 16), jnp.float32),
            jax.ShapeDtypeStruct((1, 16), jnp.float32),
        ],
    )(x, W, SD)


_MESH = plsc.VectorSubcoreMesh(core_axis_name="c", subcore_axis_name="s")


@functools.partial(
    pl.kernel,
    mesh=_MESH,
    out_type=[
        jax.ShapeDtypeStruct((E, H), jnp.float32),
        jax.ShapeDtypeStruct((2 * N, H), jnp.float32),
    ],
    scratch_types=[
        pltpu.VMEM((128,), jnp.int32),
        pltpu.VMEM((1, 128), jnp.int32),
        pltpu.VMEM((128, 16), jnp.float32),
        pltpu.VMEM((128, 16), jnp.float32),
        pltpu.VMEM((128, H), jnp.float32),
        pltpu.VMEM((16,), jnp.float32),
        pltpu.VMEM_SHARED((N, H), jnp.float32),
        pltpu.SemaphoreType.DMA,
    ],
)
def _edge_pass1(scores_hbm, gmax_hbm, src_hbm, dst_hbm, w_hbm, ssum_hbm,
                idx_s, idx_d, a_buf, b_buf, w_buf, g_buf, ssum_sh, sem):
    c = lax.axis_index("c")
    s = lax.axis_index("s")
    wid = s * NC + c

    iota = lax.iota(jnp.int32, 16)
    rb2 = iota >> 3
    col_a = iota & 7
    col_b = col_a + 8
    zero16 = jnp.zeros((16,), jnp.float32)

    def _z(j, _):
        plsc.store_scatter(w_buf, [rb2 + 2 * j, col_a], zero16)
        return 0
    lax.fori_loop(0, 64, _z, 0)
    row0 = s * ROWS_PER_TILE
    for j in range(5):
        pltpu.sync_copy(w_buf.at[pl.ds(0, 125)],
                        ssum_sh.at[pl.ds(row0 + 125 * j, 125)])
    plsc.subcore_barrier()

    pltpu.sync_copy(gmax_hbm, g_buf)
    g2 = g_buf[...]

    q, r = NCHUNK // (NC * NS), NCHUNK % (NC * NS)
    count = q + jnp.where(wid < r, 1, 0)
    chunk0 = q * wid + jnp.minimum(wid, r)

    def chunk_body(i, _):
        base = (chunk0 + i) * 128
        pltpu.sync_copy(src_hbm.at[pl.ds(base, 128)], idx_s)
        pltpu.sync_copy(dst_hbm.at[pl.ds(base, 128)], idx_d.at[0])
        pltpu.async_copy(scores_hbm.at[idx_s], a_buf, sem).wait()
        pltpu.async_copy(scores_hbm.at[idx_d.at[0]], b_buf, sem).wait()

        def pair(j, _):
            rows = rb2 + 2 * j
            a = plsc.load_gather(a_buf, [rows, col_a])
            b = plsc.load_gather(b_buf, [rows, col_b])
            t = a + b
            attn = jnp.maximum(t, 0.2 * t)
            u = g2 + b
            cc = jnp.maximum(u, 0.2 * u)
            plsc.store_scatter(w_buf, [rows, col_a], jnp.exp(attn - cc))
            return 0
        lax.fori_loop(0, 64, pair, 0)

        pltpu.sync_copy(w_buf, w_hbm.at[pl.ds(base, 128)])
        pltpu.sync_copy(w_buf, ssum_sh.at[idx_d.at[0]], add=True)
        return 0
    lax.fori_loop(0, count, chunk_body, 0)

    plsc.subcore_barrier()
    out_row0 = c * N + s * ROWS_PER_TILE
    pltpu.sync_copy(ssum_sh.at[pl.ds(s * ROWS_PER_TILE, ROWS_PER_TILE)],
                    ssum_hbm.at[pl.ds(out_row0, ROWS_PER_TILE)])


@functools.partial(
    pl.kernel,
    mesh=_MESH,
    out_type=jax.ShapeDtypeStruct((2, N, 128), jnp.float32),
    scratch_types=[
        pltpu.VMEM((1, 128), jnp.int32),
        pltpu.VMEM((128,), jnp.int32),
        pltpu.VMEM((128,), jnp.int32),
        pltpu.VMEM((128,), jnp.int32),
        pltpu.VMEM((128, 128), jnp.float32),
        pltpu.VMEM((128, H), jnp.float32),
        pltpu.VMEM((128, H), jnp.float32),
        pltpu.VMEM((128, H), jnp.float32),
        pltpu.VMEM((128, 4), jnp.float32),
        pltpu.VMEM_SHARED((N, 128), jnp.float32),
        pltpu.SemaphoreType.DMA,
    ],
)
def _aggregate(xl2_hbm, w_hbm, ssum_hbm, src_hbm, dst_hbm, out_hbm,
               idx_s, idx_d, idx_xl, idx_p1, r_buf, w_slab, p0_buf, p1_buf,
               vals_buf, acc_sh, sem):
    c = lax.axis_index("c")
    s = lax.axis_index("s")

    iota = lax.iota(jnp.int32, 16)
    zero16 = jnp.zeros((16,), jnp.float32)
    col16 = [iota + 16 * k for k in range(8)]
    vcol = [jnp.full((16,), hh, jnp.int32) for hh in range(4)]
    wcol = [jnp.zeros((16,), jnp.int32) + (c * 4 + hh) for hh in range(4)]

    def _z(e, _):
        es = jnp.zeros((16,), jnp.int32) + e
        for k in range(8):
            plsc.store_scatter(r_buf, [es, col16[k]], zero16)
        return 0
    lax.fori_loop(0, 128, _z, 0)
    row0 = s * ROWS_PER_TILE
    for j in range(5):
        pltpu.sync_copy(r_buf.at[pl.ds(0, 125)],
                        acc_sh.at[pl.ds(row0 + 125 * j, 125)])
    plsc.subcore_barrier()

    q, r = NCHUNK // NS, NCHUNK % NS
    count = q + jnp.where(s < r, 1, 0)
    chunk0 = q * s + jnp.minimum(s, r)
    cN = c * N

    def chunk_body(i, _):
        base = (chunk0 + i) * 128
        pltpu.sync_copy(src_hbm.at[pl.ds(base, 128)], idx_s.at[0])
        pltpu.sync_copy(dst_hbm.at[pl.ds(base, 128)], idx_d)

        def mkidx(g, _):
            v = idx_d[pl.ds(16 * g, 16)]
            idx_xl[pl.ds(16 * g, 16)] = v + cN
            idx_p1[pl.ds(16 * g, 16)] = v + N
            return 0
        lax.fori_loop(0, 8, mkidx, 0)

        pltpu.async_copy(xl2_hbm.at[idx_xl], r_buf, sem).wait()
        pltpu.async_copy(ssum_hbm.at[idx_d], p0_buf, sem).wait()
        pltpu.async_copy(ssum_hbm.at[idx_p1], p1_buf, sem).wait()
        pltpu.sync_copy(w_hbm.at[pl.ds(base, 128)], w_slab)

        def grp(g, _):
            rows = iota + 16 * g
            for hh in range(4):
                wv = plsc.load_gather(w_slab, [rows, wcol[hh]])
                p0 = plsc.load_gather(p0_buf, [rows, wcol[hh]])
                p1 = plsc.load_gather(p1_buf, [rows, wcol[hh]])
                plsc.store_scatter(vals_buf, [rows, vcol[hh]], wv / (p0 + p1))
            return 0
        lax.fori_loop(0, 8, grp, 0)

        def scale(e, _):
            es = jnp.zeros((16,), jnp.int32) + e
            sv = [plsc.load_gather(vals_buf, [es, vcol[hh]]) for hh in range(4)]
            for k in range(8):
                rv = plsc.load_gather(r_buf, [es, col16[k]])
                plsc.store_scatter(r_buf, [es, col16[k]], rv * sv[k // 2])
            return 0
        lax.fori_loop(0, 128, scale, 0)

        pltpu.sync_copy(r_buf, acc_sh.at[idx_s.at[0]], add=True)
        return 0
    lax.fori_loop(0, count, chunk_body, 0)

    plsc.subcore_barrier()
    for j in range(5):
        rr = row0 + 125 * j
        pltpu.sync_copy(acc_sh.at[pl.ds(rr, 125)],
                        out_hbm.at[c, pl.ds(rr, 125)])


def kernel(x, edge_indices, W, src_attn, dst_attn):
    eye8 = jnp.eye(H, dtype=jnp.float32)
    SA = (src_attn[0][:, :, None] * eye8[:, None, :]).reshape(HD, H)
    DA = (dst_attn[0][:, :, None] * eye8[:, None, :]).reshape(HD, H)
    SD = jnp.concatenate([SA, DA], axis=1)

    xlp, scores, gmax = _project(x, W, SD)
    xl2 = xlp.reshape(2 * N, 128)
    src = edge_indices[0]
    dst = edge_indices[1]

    w, ssum2 = _edge_pass1(scores, gmax.reshape(16), src, dst)
    out_pairs = _aggregate(xl2, w, ssum2, src, dst)
    return out_pairs.transpose(1, 0, 2).reshape(N, HD)


# speedup vs baseline: 28.8816x; 28.8816x over previous
import functools

import jax
import jax.numpy as jnp
from jax import lax
from jax.experimental import pallas as pl
from jax.experimental.pallas import tpu as pltpu
from jax.experimental.pallas import tpu_sc as plsc

N = 10000
E = 160000
DIN = 256
H = 8
DH = 32
HD = H * DH

BN = 400
NCHUNK = E // 128
NC = 2
NS = 16
ROW_Q = 624


def _project_body(x_ref, w_ref, sd_ref, xlp_ref, ssd_ref, sdd_ref, gmax_ref):
    i = pl.program_id(0)
    y = lax.dot_general(x_ref[...], w_ref[...],
                        dimension_numbers=(((1,), (1,)), ((), ())),
                        preferred_element_type=jnp.float32)
    scores = jnp.dot(y, sd_ref[...], preferred_element_type=jnp.float32)
    xlp_ref[...] = jnp.stack([y[:, :128], y[:, 128:]], axis=0)
    pad = jnp.zeros((BN, 128 - 16), jnp.float32)
    ssd_ref[...] = jnp.concatenate([scores, pad], axis=1)
    ds_part = scores[:, H:]
    sdd_ref[...] = jnp.concatenate([ds_part, ds_part, pad], axis=1)
    m = jnp.max(scores[:, :H], axis=0)
    g16 = jnp.concatenate([m, m])[None, :]

    @pl.when(i == 0)
    def _():
        gmax_ref[...] = g16

    @pl.when(i != 0)
    def _():
        gmax_ref[...] = jnp.maximum(gmax_ref[...], g16)


def _project(x, W, SD):
    return pl.pallas_call(
        _project_body,
        grid=(N // BN,),
        in_specs=[
            pl.BlockSpec((BN, DIN), lambda i: (i, 0)),
            pl.BlockSpec((HD, DIN), lambda i: (0, 0)),
            pl.BlockSpec((DIN, 16), lambda i: (0, 0)),
        ],
        out_specs=[
            pl.BlockSpec((2, BN, 128), lambda i: (0, i, 0)),
            pl.BlockSpec((BN, 128), lambda i: (i, 0)),
            pl.BlockSpec((BN, 128), lambda i: (i, 0)),
            pl.BlockSpec((1, 16), lambda i: (0, 0)),
        ],
        out_shape=[
            jax.ShapeDtypeStruct((2, N, 128), jnp.float32),
            jax.ShapeDtypeStruct((N, 128), jnp.float32),
            jax.ShapeDtypeStruct((N, 128), jnp.float32),
            jax.ShapeDtypeStruct((1, 16), jnp.float32),
        ],
    )(x, W, SD)


def _normalize_body(xlp_ref, ssum_ref, exp8_ref, yl_ref):
    tot = ssum_ref[0, :, :H] + ssum_ref[1, :, :H]
    r256 = jnp.dot(1.0 / tot, exp8_ref[...],
                   preferred_element_type=jnp.float32)
    yl_ref[...] = jnp.stack(
        [xlp_ref[0] * r256[:, :128], xlp_ref[1] * r256[:, 128:]], axis=0)


def _normalize(xlp, ssum3, EXP8):
    return pl.pallas_call(
        _normalize_body,
        grid=(N // BN,),
        in_specs=[
            pl.BlockSpec((2, BN, 128), lambda i: (0, i, 0)),
            pl.BlockSpec((2, BN, 16), lambda i: (0, i, 0)),
            pl.BlockSpec((H, HD), lambda i: (0, 0)),
        ],
        out_specs=pl.BlockSpec((2, BN, 128), lambda i: (0, i, 0)),
        out_shape=jax.ShapeDtypeStruct((2, N, 128), jnp.float32),
    )(xlp, ssum3, EXP8)


def _lane_gather(v, idx16):
    dnums = lax.GatherDimensionNumbers(
        offset_dims=(), collapsed_slice_dims=(0,), start_index_map=(0,))
    return lax.gather(v, idx16[:, None], dnums, (1,),
                      mode=lax.GatherScatterMode.PROMISE_IN_BOUNDS)


_MESH = plsc.VectorSubcoreMesh(core_axis_name="c", subcore_axis_name="s")


@functools.partial(
    pl.kernel,
    mesh=_MESH,
    out_type=[
        jax.ShapeDtypeStruct((E, 16), jnp.float32),
        jax.ShapeDtypeStruct((2 * N, 16), jnp.float32),
    ],
    scratch_types=[
        pltpu.VMEM((128,), jnp.int32),
        pltpu.VMEM((1, 128), jnp.int32),
        pltpu.VMEM((128, 128), jnp.float32),
        pltpu.VMEM((128, 128), jnp.float32),
        pltpu.VMEM((128, 16), jnp.float32),
        pltpu.VMEM((16,), jnp.float32),
        pltpu.VMEM_SHARED((N, 16), jnp.float32),
        pltpu.SemaphoreType.DMA,
    ],
)
def _edge_pass1(ssd_hbm, sdd_hbm, gmax_hbm, src_hbm, dst_hbm, w_hbm, ssum_hbm,
                idx_s, idx_d, a_buf, b_buf, w_buf, g_buf, ssum_sh, sem):
    c = lax.axis_index("c")
    s = lax.axis_index("s")
    wid = s * NC + c

    zero16 = jnp.zeros((16,), jnp.float32)

    def _z(j, _):
        w_buf[j] = zero16
        return 0
    lax.fori_loop(0, 128, _z, 0)
    row0 = s * ROW_Q

    @pl.when(s < NS - 1)
    def _():
        for j in range(6):
            pltpu.sync_copy(w_buf.at[pl.ds(0, 104)],
                            ssum_sh.at[pl.ds(row0 + 104 * j, 104)])

    @pl.when(s == NS - 1)
    def _():
        for j in range(5):
            pltpu.sync_copy(w_buf, ssum_sh.at[pl.ds(row0 + 128 * j, 128)])
    plsc.subcore_barrier()

    pltpu.sync_copy(gmax_hbm, g_buf)
    g2 = g_buf[...]

    q, r = NCHUNK // (NC * NS), NCHUNK % (NC * NS)
    count = q + jnp.where(wid < r, 1, 0)
    chunk0 = q * wid + jnp.minimum(wid, r)

    def chunk_body(i, _):
        base = pl.multiple_of((chunk0 + i) * 128, 128)
        pltpu.sync_copy(src_hbm.at[pl.ds(base, 128)], idx_s)
        pltpu.sync_copy(dst_hbm.at[pl.ds(base, 128)], idx_d.at[0])
        pltpu.async_copy(ssd_hbm.at[idx_s], a_buf, sem).wait()
        pltpu.async_copy(sdd_hbm.at[idx_d.at[0]], b_buf, sem).wait()

        def edge(e, _):
            a = a_buf[e, pl.ds(0, 16)]
            b = b_buf[e, pl.ds(0, 16)]
            t = a + b
            attn = jnp.maximum(t, 0.2 * t)
            u = g2 + b
            cc = jnp.maximum(u, 0.2 * u)
            w_buf[e] = jnp.exp(attn - cc)
            return 0
        lax.fori_loop(0, 128, edge, 0)

        pltpu.sync_copy(w_buf, w_hbm.at[pl.ds(base, 128)])
        pltpu.sync_copy(w_buf, ssum_sh.at[idx_d.at[0]], add=True)
        return 0
    lax.fori_loop(0, count, chunk_body, 0)

    plsc.subcore_barrier()
    out_row0 = c * N + row0

    @pl.when(s < NS - 1)
    def _():
        for j in range(6):
            pltpu.sync_copy(ssum_sh.at[pl.ds(row0 + 104 * j, 104)],
                            w_buf.at[pl.ds(0, 104)])
            pltpu.sync_copy(w_buf.at[pl.ds(0, 104)],
                            ssum_hbm.at[pl.ds(out_row0 + 104 * j, 104)])

    @pl.when(s == NS - 1)
    def _():
        for j in range(5):
            pltpu.sync_copy(ssum_sh.at[pl.ds(row0 + 128 * j, 128)], w_buf)
            pltpu.sync_copy(w_buf,
                            ssum_hbm.at[pl.ds(out_row0 + 128 * j, 128)])


@functools.partial(
    pl.kernel,
    mesh=_MESH,
    out_type=jax.ShapeDtypeStruct((2, N, 128), jnp.float32),
    scratch_types=[
        pltpu.VMEM((1, 128), jnp.int32),
        pltpu.VMEM((128,), jnp.int32),
        pltpu.VMEM((128,), jnp.int32),
        pltpu.VMEM((128, 128), jnp.float32),
        pltpu.VMEM((128, 16), jnp.float32),
        pltpu.VMEM_SHARED((N, 128), jnp.float32),
        pltpu.SemaphoreType.DMA,
    ],
)
def _aggregate(yl2_hbm, w_hbm, src_hbm, dst_hbm, out_hbm,
               idx_s, idx_d, idx_xl, r_buf, w_slab, acc_sh, sem):
    c = lax.axis_index("c")
    s = lax.axis_index("s")

    zero16 = jnp.zeros((16,), jnp.float32)
    hsel = [jnp.zeros((16,), jnp.int32) + (c * 4 + hh) for hh in range(4)]

    def _z(e, _):
        for k in range(8):
            r_buf[e, pl.ds(16 * k, 16)] = zero16
        return 0
    lax.fori_loop(0, 128, _z, 0)
    row0 = s * ROW_Q

    @pl.when(s < NS - 1)
    def _():
        for j in range(6):
            pltpu.sync_copy(r_buf.at[pl.ds(0, 104)],
                            acc_sh.at[pl.ds(row0 + 104 * j, 104)])

    @pl.when(s == NS - 1)
    def _():
        for j in range(5):
            pltpu.sync_copy(r_buf, acc_sh.at[pl.ds(row0 + 128 * j, 128)])
    plsc.subcore_barrier()

    q, r = NCHUNK // NS, NCHUNK % NS
    count = q + jnp.where(s < r, 1, 0)
    chunk0 = q * s + jnp.minimum(s, r)
    cN = c * N

    def chunk_body(i, _):
        base = (chunk0 + i) * 128
        pltpu.sync_copy(src_hbm.at[pl.ds(base, 128)], idx_s.at[0])
        pltpu.sync_copy(dst_hbm.at[pl.ds(base, 128)], idx_d)

        def mkidx(g, _):
            v = idx_d[pl.ds(16 * g, 16)]
            idx_xl[pl.ds(16 * g, 16)] = v + cN
            return 0
        lax.fori_loop(0, 8, mkidx, 0)

        pltpu.async_copy(yl2_hbm.at[idx_xl], r_buf, sem).wait()
        pltpu.sync_copy(w_hbm.at[pl.ds(base, 128)], w_slab)

        def scale(e, _):
            vals = w_slab[e]
            for hh in range(4):
                sv = _lane_gather(vals, hsel[hh])
                for k in (2 * hh, 2 * hh + 1):
                    rv = r_buf[e, pl.ds(16 * k, 16)]
                    r_buf[e, pl.ds(16 * k, 16)] = rv * sv
            return 0
        lax.fori_loop(0, 128, scale, 0)

        pltpu.sync_copy(r_buf, acc_sh.at[idx_s.at[0]], add=True)
        return 0
    lax.fori_loop(0, count, chunk_body, 0)

    plsc.subcore_barrier()

    @pl.when(s < NS - 1)
    def _():
        for j in range(6):
            rr = row0 + 104 * j
            pltpu.sync_copy(acc_sh.at[pl.ds(rr, 104)], r_buf.at[pl.ds(0, 104)])
            pltpu.sync_copy(r_buf.at[pl.ds(0, 104)],
                            out_hbm.at[c, pl.ds(rr, 104)])

    @pl.when(s == NS - 1)
    def _():
        for j in range(5):
            rr = row0 + 128 * j
            pltpu.sync_copy(acc_sh.at[pl.ds(rr, 128)], r_buf)
            pltpu.sync_copy(r_buf, out_hbm.at[c, pl.ds(rr, 128)])


def kernel(x, edge_indices, W, src_attn, dst_attn):
    eye8 = jnp.eye(H, dtype=jnp.float32)
    SA = (src_attn[0][:, :, None] * eye8[:, None, :]).reshape(HD, H)
    DA = (dst_attn[0][:, :, None] * eye8[:, None, :]).reshape(HD, H)
    SD = jnp.concatenate([SA, DA], axis=1)

    EXP8 = jnp.repeat(eye8, DH, axis=1)

    xlp, scores_sd, scores_dd, gmax = _project(x, W, SD)
    src = edge_indices[0]
    dst = edge_indices[1]

    w, ssum2 = _edge_pass1(scores_sd, scores_dd, gmax.reshape(16), src, dst)
    yl2 = _normalize(xlp, ssum2.reshape(2, N, 16), EXP8).reshape(2 * N, 128)
    out_pairs = _aggregate(yl2, w, src, dst)
    return out_pairs.transpose(1, 0, 2).reshape(N, HD)


# speedup vs baseline: 29.1180x; 1.0082x over previous
import functools

import jax
import jax.numpy as jnp
from jax import lax
from jax.experimental import pallas as pl
from jax.experimental.pallas import tpu as pltpu
from jax.experimental.pallas import tpu_sc as plsc

N = 10000
E = 160000
DIN = 256
H = 8
DH = 32
HD = H * DH

BN = 400
NCHUNK = E // 128
NC = 2
NS = 16
ROW_Q = 624


def _project_body(x_ref, w_ref, sd_ref, xlp_ref, ssd_ref, sdd_ref, gmax_ref):
    i = pl.program_id(0)
    y = lax.dot_general(x_ref[...], w_ref[...],
                        dimension_numbers=(((1,), (1,)), ((), ())),
                        preferred_element_type=jnp.float32)
    scores = jnp.dot(y, sd_ref[...], preferred_element_type=jnp.float32)
    xlp_ref[...] = jnp.stack([y[:, :128], y[:, 128:]], axis=0)
    pad = jnp.zeros((BN, 128 - 16), jnp.float32)
    ssd_ref[...] = jnp.concatenate([scores, pad], axis=1)
    ds_part = scores[:, H:]
    sdd_ref[...] = jnp.concatenate([ds_part, ds_part, pad], axis=1)
    m = jnp.max(scores[:, :H], axis=0)
    g16 = jnp.concatenate([m, m])[None, :]

    @pl.when(i == 0)
    def _():
        gmax_ref[...] = g16

    @pl.when(i != 0)
    def _():
        gmax_ref[...] = jnp.maximum(gmax_ref[...], g16)


def _project(x, W, SD):
    return pl.pallas_call(
        _project_body,
        grid=(N // BN,),
        in_specs=[
            pl.BlockSpec((BN, DIN), lambda i: (i, 0)),
            pl.BlockSpec((HD, DIN), lambda i: (0, 0)),
            pl.BlockSpec((DIN, 16), lambda i: (0, 0)),
        ],
        out_specs=[
            pl.BlockSpec((2, BN, 128), lambda i: (0, i, 0)),
            pl.BlockSpec((BN, 128), lambda i: (i, 0)),
            pl.BlockSpec((BN, 128), lambda i: (i, 0)),
            pl.BlockSpec((1, 16), lambda i: (0, 0)),
        ],
        out_shape=[
            jax.ShapeDtypeStruct((2, N, 128), jnp.float32),
            jax.ShapeDtypeStruct((N, 128), jnp.float32),
            jax.ShapeDtypeStruct((N, 128), jnp.float32),
            jax.ShapeDtypeStruct((1, 16), jnp.float32),
        ],
    )(x, W, SD)


def _normalize_body(xlp_ref, ssum_ref, exp8_ref, yl_ref):
    tot = ssum_ref[0, :, :H] + ssum_ref[1, :, :H]
    r256 = jnp.dot(1.0 / tot, exp8_ref[...],
                   preferred_element_type=jnp.float32)
    yl_ref[...] = jnp.stack(
        [xlp_ref[0] * r256[:, :128], xlp_ref[1] * r256[:, 128:]], axis=0)


def _normalize(xlp, ssum3, EXP8):
    return pl.pallas_call(
        _normalize_body,
        grid=(N // BN,),
        in_specs=[
            pl.BlockSpec((2, BN, 128), lambda i: (0, i, 0)),
            pl.BlockSpec((2, BN, 16), lambda i: (0, i, 0)),
            pl.BlockSpec((H, HD), lambda i: (0, 0)),
        ],
        out_specs=pl.BlockSpec((2, BN, 128), lambda i: (0, i, 0)),
        out_shape=jax.ShapeDtypeStruct((2, N, 128), jnp.float32),
    )(xlp, ssum3, EXP8)


def _lane_gather(v, idx16):
    dnums = lax.GatherDimensionNumbers(
        offset_dims=(), collapsed_slice_dims=(0,), start_index_map=(0,))
    return lax.gather(v, idx16[:, None], dnums, (1,),
                      mode=lax.GatherScatterMode.PROMISE_IN_BOUNDS)


_MESH = plsc.VectorSubcoreMesh(core_axis_name="c", subcore_axis_name="s")


@functools.partial(
    pl.kernel,
    mesh=_MESH,
    out_type=[
        jax.ShapeDtypeStruct((E, 16), jnp.float32),
        jax.ShapeDtypeStruct((2 * N, 16), jnp.float32),
    ],
    scratch_types=[
        pltpu.VMEM((128,), jnp.int32),
        pltpu.VMEM((1, 128), jnp.int32),
        pltpu.VMEM((128, 128), jnp.float32),
        pltpu.VMEM((128, 128), jnp.float32),
        pltpu.VMEM((128, 16), jnp.float32),
        pltpu.VMEM((16,), jnp.float32),
        pltpu.VMEM_SHARED((N, 16), jnp.float32),
        pltpu.SemaphoreType.DMA,
    ],
)
def _edge_pass1(ssd_hbm, sdd_hbm, gmax_hbm, src_hbm, dst_hbm, w_hbm, ssum_hbm,
                idx_s, idx_d, a_buf, b_buf, w_buf, g_buf, ssum_sh, sem):
    c = lax.axis_index("c")
    s = lax.axis_index("s")
    wid = s * NC + c

    zero16 = jnp.zeros((16,), jnp.float32)

    def _z(j, _):
        w_buf[j] = zero16
        return 0
    lax.fori_loop(0, 128, _z, 0)
    row0 = s * ROW_Q

    @pl.when(s < NS - 1)
    def _():
        for j in range(6):
            pltpu.sync_copy(w_buf.at[pl.ds(0, 104)],
                            ssum_sh.at[pl.ds(row0 + 104 * j, 104)])

    @pl.when(s == NS - 1)
    def _():
        for j in range(5):
            pltpu.sync_copy(w_buf, ssum_sh.at[pl.ds(row0 + 128 * j, 128)])
    plsc.subcore_barrier()

    pltpu.sync_copy(gmax_hbm, g_buf)
    g2 = g_buf[...]

    q, r = NCHUNK // (NC * NS), NCHUNK % (NC * NS)
    count = q + jnp.where(wid < r, 1, 0)
    chunk0 = q * wid + jnp.minimum(wid, r)

    def chunk_body(i, _):
        base = pl.multiple_of((chunk0 + i) * 128, 128)
        pltpu.sync_copy(src_hbm.at[pl.ds(base, 128)], idx_s)
        pltpu.sync_copy(dst_hbm.at[pl.ds(base, 128)], idx_d.at[0])
        pltpu.async_copy(ssd_hbm.at[idx_s], a_buf, sem).wait()
        pltpu.async_copy(sdd_hbm.at[idx_d.at[0]], b_buf, sem).wait()

        def edge(j, _):
            for u_ in range(2):
                e = 2 * j + u_
                a = a_buf[e, pl.ds(0, 16)]
                b = b_buf[e, pl.ds(0, 16)]
                t = a + b
                attn = jnp.maximum(t, 0.2 * t)
                u = g2 + b
                cc = jnp.maximum(u, 0.2 * u)
                w_buf[e] = jnp.exp(attn - cc)
            return 0
        lax.fori_loop(0, 64, edge, 0)

        pltpu.sync_copy(w_buf, w_hbm.at[pl.ds(base, 128)])
        pltpu.sync_copy(w_buf, ssum_sh.at[idx_d.at[0]], add=True)
        return 0
    lax.fori_loop(0, count, chunk_body, 0)

    plsc.subcore_barrier()
    out_row0 = c * N + row0

    @pl.when(s < NS - 1)
    def _():
        for j in range(6):
            pltpu.sync_copy(ssum_sh.at[pl.ds(row0 + 104 * j, 104)],
                            w_buf.at[pl.ds(0, 104)])
            pltpu.sync_copy(w_buf.at[pl.ds(0, 104)],
                            ssum_hbm.at[pl.ds(out_row0 + 104 * j, 104)])

    @pl.when(s == NS - 1)
    def _():
        for j in range(5):
            pltpu.sync_copy(ssum_sh.at[pl.ds(row0 + 128 * j, 128)], w_buf)
            pltpu.sync_copy(w_buf,
                            ssum_hbm.at[pl.ds(out_row0 + 128 * j, 128)])


@functools.partial(
    pl.kernel,
    mesh=_MESH,
    out_type=jax.ShapeDtypeStruct((2, N, 128), jnp.float32),
    scratch_types=[
        pltpu.VMEM((1, 128), jnp.int32),
        pltpu.VMEM((128,), jnp.int32),
        pltpu.VMEM((128,), jnp.int32),
        pltpu.VMEM((128, 128), jnp.float32),
        pltpu.VMEM((128, 16), jnp.float32),
        pltpu.VMEM_SHARED((N, 128), jnp.float32),
        pltpu.SemaphoreType.DMA,
    ],
)
def _aggregate(yl2_hbm, w_hbm, src_hbm, dst_hbm, out_hbm,
               idx_s, idx_d, idx_xl, r_buf, w_slab, acc_sh, sem):
    c = lax.axis_index("c")
    s = lax.axis_index("s")

    zero16 = jnp.zeros((16,), jnp.float32)
    hsel = [jnp.zeros((16,), jnp.int32) + (c * 4 + hh) for hh in range(4)]

    def _z(e, _):
        for k in range(8):
            r_buf[e, pl.ds(16 * k, 16)] = zero16
        return 0
    lax.fori_loop(0, 128, _z, 0)
    row0 = s * ROW_Q

    @pl.when(s < NS - 1)
    def _():
        for j in range(6):
            pltpu.sync_copy(r_buf.at[pl.ds(0, 104)],
                            acc_sh.at[pl.ds(row0 + 104 * j, 104)])

    @pl.when(s == NS - 1)
    def _():
        for j in range(5):
            pltpu.sync_copy(r_buf, acc_sh.at[pl.ds(row0 + 128 * j, 128)])
    plsc.subcore_barrier()

    q, r = NCHUNK // NS, NCHUNK % NS
    count = q + jnp.where(s < r, 1, 0)
    chunk0 = q * s + jnp.minimum(s, r)
    cN = c * N

    def chunk_body(i, _):
        base = (chunk0 + i) * 128
        pltpu.sync_copy(src_hbm.at[pl.ds(base, 128)], idx_s.at[0])
        pltpu.sync_copy(dst_hbm.at[pl.ds(base, 128)], idx_d)

        def mkidx(g, _):
            v = idx_d[pl.ds(16 * g, 16)]
            idx_xl[pl.ds(16 * g, 16)] = v + cN
            return 0
        lax.fori_loop(0, 8, mkidx, 0)

        pltpu.async_copy(yl2_hbm.at[idx_xl], r_buf, sem).wait()
        pltpu.sync_copy(w_hbm.at[pl.ds(base, 128)], w_slab)

        def scale(j, _):
            for u_ in range(2):
                e = 2 * j + u_
                vals = w_slab[e]
                for hh in range(4):
                    sv = _lane_gather(vals, hsel[hh])
                    for k in (2 * hh, 2 * hh + 1):
                        rv = r_buf[e, pl.ds(16 * k, 16)]
                        r_buf[e, pl.ds(16 * k, 16)] = rv * sv
            return 0
        lax.fori_loop(0, 64, scale, 0)

        pltpu.sync_copy(r_buf, acc_sh.at[idx_s.at[0]], add=True)
        return 0
    lax.fori_loop(0, count, chunk_body, 0)

    plsc.subcore_barrier()

    @pl.when(s < NS - 1)
    def _():
        for j in range(6):
            rr = row0 + 104 * j
            pltpu.sync_copy(acc_sh.at[pl.ds(rr, 104)], r_buf.at[pl.ds(0, 104)])
            pltpu.sync_copy(r_buf.at[pl.ds(0, 104)],
                            out_hbm.at[c, pl.ds(rr, 104)])

    @pl.when(s == NS - 1)
    def _():
        for j in range(5):
            rr = row0 + 128 * j
            pltpu.sync_copy(acc_sh.at[pl.ds(rr, 128)], r_buf)
            pltpu.sync_copy(r_buf, out_hbm.at[c, pl.ds(rr, 128)])


def kernel(x, edge_indices, W, src_attn, dst_attn):
    eye8 = jnp.eye(H, dtype=jnp.float32)
    SA = (src_attn[0][:, :, None] * eye8[:, None, :]).reshape(HD, H)
    DA = (dst_attn[0][:, :, None] * eye8[:, None, :]).reshape(HD, H)
    SD = jnp.concatenate([SA, DA], axis=1)

    EXP8 = jnp.repeat(eye8, DH, axis=1)

    xlp, scores_sd, scores_dd, gmax = _project(x, W, SD)
    src = edge_indices[0]
    dst = edge_indices[1]

    w, ssum2 = _edge_pass1(scores_sd, scores_dd, gmax.reshape(16), src, dst)
    yl2 = _normalize(xlp, ssum2.reshape(2, N, 16), EXP8).reshape(2 * N, 128)
    out_pairs = _aggregate(yl2, w, src, dst)
    return out_pairs.transpose(1, 0, 2).reshape(N, HD)
